# Initial kernel scaffold; baseline (speedup 1.0000x reference)
import jax
import jax.numpy as jnp
from jax.experimental import pallas as pl


def kernel(x, edge_index, edge_attr, batch, params):
    raise NotImplementedError("write your pallas kernel here")


# speedup vs baseline: 1.0002x; 1.0002x over previous
import jax
import jax.numpy as jnp
from jax.experimental import pallas as pl


def _mlp(p, h):
    h = jnp.dot(h, p['W1']) + p['b1']
    h = jax.nn.relu(h)
    return jnp.dot(h, p['W2']) + p['b2']


def _bn(h, g, b, eps=1e-5):
    m = jnp.mean(h, axis=0)
    v = jnp.var(h, axis=0)
    return g * (h - m) / jnp.sqrt(v + eps) + b


def _gine(x, edge_index, edge_attr, mlp_p, eps=0.0):
    src = edge_index[0]
    dst = edge_index[1]
    msg = jax.nn.relu(x[src] + edge_attr)
    aggr = jax.ops.segment_sum(msg, dst, num_segments=x.shape[0])
    return _mlp(mlp_p, (1.0 + eps) * x + aggr)


def _identity_kernel(x_ref, o_ref):
    o_ref[...] = x_ref[...]


def kernel(x, edge_index, edge_attr, batch, params):
    ea1 = jnp.dot(edge_attr, params['ee_W']) + params['ee_b']
    h = _gine(x, edge_index, ea1, params['mlp1'])
    h = _bn(h, params['bn1_g'], params['bn1_b'])
    h = jax.nn.relu(h)
    ead = jnp.dot(ea1, params['el_W']) + params['el_b']
    h = _gine(h, edge_index, ead, params['mlp2'])
    h = _bn(h, params['bn2_g'], params['bn2_b'])
    h = jax.nn.relu(h)
    h = _gine(h, edge_index, ead, params['mlp3'])
    h = _bn(h, params['bn3_g'], params['bn3_b'])
    h = jax.nn.relu(h)
    sums = jax.ops.segment_sum(h, batch, num_segments=128)
    cnts = jax.ops.segment_sum(jnp.ones((h.shape[0],), jnp.float32), batch, num_segments=128)
    pooled = sums / jnp.maximum(cnts, 1.0)[:, None]
    z = jax.nn.relu(jnp.dot(pooled, params['h_W1']) + params['h_b1'])
    out = jnp.dot(z, params['h_W2']) + params['h_b2']
    return pl.pallas_call(
        _identity_kernel,
        out_shape=jax.ShapeDtypeStruct(out.shape, out.dtype),
    )(out)


# speedup vs baseline: 2.2463x; 2.2459x over previous
import functools

import jax
import jax.numpy as jnp
from jax import lax
from jax.experimental import pallas as pl
from jax.experimental.pallas import tpu as pltpu
from jax.experimental.pallas import tpu_sc as plsc

N = 100000
E = 1600000
H = 64
G = 128
L = 16
EB = 8000
NB = 1000
SC_B = 800
NP = 102400
NPT = NP // 16
ZR = 800


def _edge_prep_body(ea_ref, eeW_ref, eeb_ref, elW_ref, elb_ref,
                    ea1_ref, ead_ref):
    ea1 = jnp.dot(ea_ref[...], eeW_ref[...],
                  preferred_element_type=jnp.float32) + eeb_ref[...]
    ea1_ref[...] = ea1
    ead = jnp.dot(ea1, elW_ref[...],
                  preferred_element_type=jnp.float32) + elb_ref[...]
    ead_ref[...] = jnp.stack([ead[:, c * L:(c + 1) * L] for c in range(4)])


def _edge_prep(edge_attr, eeW, eeb, elW, elb):
    grid = E // EB
    return pl.pallas_call(
        _edge_prep_body,
        grid=(grid,),
        in_specs=[
            pl.BlockSpec((EB, 4), lambda i: (i, 0)),
            pl.BlockSpec((4, L), lambda i: (0, 0)),
            pl.BlockSpec((1, L), lambda i: (0, 0)),
            pl.BlockSpec((L, H), lambda i: (0, 0)),
            pl.BlockSpec((1, H), lambda i: (0, 0)),
        ],
        out_specs=[
            pl.BlockSpec((EB, L), lambda i: (i, 0)),
            pl.BlockSpec((4, EB, L), lambda i: (0, i, 0)),
        ],
        out_shape=[
            jax.ShapeDtypeStruct((E, L), jnp.float32),
            jax.ShapeDtypeStruct((4, E, L), jnp.float32),
        ],
    )(edge_attr, eeW, eeb, elW, elb)


def _make_sc_gine(C):
    mesh = plsc.VectorSubcoreMesh(core_axis_name="c", subcore_axis_name="s")
    n_out = 4 if C == 4 else 2
    Et = E // 16 if C == 4 else E // 32
    nblk, rem = divmod(Et, SC_B)
    npass = 2 if C == 4 else 1

    @functools.partial(
        pl.kernel,
        mesh=mesh,
        compiler_params=pltpu.CompilerParams(use_tc_tiling_on_sc=False),
        out_type=jax.ShapeDtypeStruct((n_out, NP, L), jnp.float32),
        scratch_types=[
            pltpu.VMEM((SC_B,), jnp.int32),
            pltpu.VMEM((SC_B,), jnp.int32),
            pltpu.VMEM((SC_B, L), jnp.float32),
            pltpu.VMEM((SC_B, L), jnp.float32),
            pltpu.VMEM_SHARED((NP + 8, L), jnp.float32),
            pltpu.SemaphoreType.DMA,
        ],
    )
    def sc_gine(tab_hbm, ea_hbm, src_hbm, dst_hbm, out_hbm,
                src_v, dst_v, rows_v, ea_v, acc, sem):
        core = lax.axis_index("c")
        tile = lax.axis_index("s")
        row0 = tile * NPT

        def process_block(base, cidx, bsz, tail):
            pltpu.sync_copy(src_hbm.at[pl.ds(base, bsz)],
                            src_v.at[pl.ds(0, bsz)])
            pltpu.sync_copy(dst_hbm.at[pl.ds(base, bsz)],
                            dst_v.at[pl.ds(0, bsz)])
            if C == 4:
                offv = jnp.full((L,), cidx * N, jnp.int32)

                def addoff(j, _):
                    sl = pl.ds(j * L, L)
                    src_v[sl] = src_v[sl] + offv
                    return 0
                lax.fori_loop(0, bsz // L, addoff, 0)
            if tail:
                safe = jnp.full((L,), cidx * N, jnp.int32)
                dump = jnp.full((L,), NP, jnp.int32)
                for j in range(bsz // L, SC_B // L):
                    src_v[pl.ds(j * L, L)] = safe
                    dst_v[pl.ds(j * L, L)] = dump
            cp = pltpu.async_copy(tab_hbm.at[src_v], rows_v, sem)
            pltpu.sync_copy(ea_hbm.at[cidx, pl.ds(base, bsz)],
                            ea_v.at[pl.ds(0, bsz)])
            cp.wait()

            def body(i, _):
                rows_v[i] = jnp.maximum(rows_v[i] + ea_v[i], 0.0)
                return 0
            lax.fori_loop(0, bsz, body, 0, unroll=8)
            pltpu.sync_copy(rows_v, acc.at[dst_v], add=True)

        for p in range(npass):
            if C == 4:
                cidx = core * 2 + p
                tile_base = tile * Et
            else:
                cidx = 0
                tile_base = core * (E // 2) + tile * Et

            def zfill(j, _):
                rows_v[j] = jnp.zeros((L,), jnp.float32)
                return 0
            lax.fori_loop(0, SC_B, zfill, 0)
            for k in range(NPT // ZR):
                pltpu.sync_copy(rows_v, acc.at[pl.ds(row0 + k * ZR, ZR)])
            plsc.subcore_barrier()

            def blk(b, _):
                process_block(tile_base + b * SC_B, cidx, SC_B, False)
                return 0
            lax.fori_loop(0, nblk, blk, 0)
            if rem:
                process_block(tile_base + nblk * SC_B, cidx, rem, True)

            plsc.subcore_barrier()
            out_idx = cidx if C == 4 else core
            pltpu.sync_copy(acc.at[pl.ds(row0, NPT)],
                            out_hbm.at[out_idx, pl.ds(row0, NPT)])
            if p + 1 < npass:
                plsc.subcore_barrier()

    return sc_gine


_sc_gine_c1 = _make_sc_gine(1)
_sc_gine_c4 = _make_sc_gine(4)


def _passA1_body(x_ref, pa_ref, W1_ref, b1_ref, W2_ref, b2_ref,
                 u_ref, st_ref):
    i = pl.program_id(0)
    a = x_ref[...] + pa_ref[0] + pa_ref[1]
    t = jnp.maximum(jnp.dot(a, W1_ref[...],
                            preferred_element_type=jnp.float32)
                    + b1_ref[...], 0.0)
    u = jnp.dot(t, W2_ref[...], preferred_element_type=jnp.float32) \
        + b2_ref[...]
    u_ref[...] = u

    @pl.when(i == 0)
    def _():
        st_ref[...] = jnp.zeros_like(st_ref)
    st_ref[...] += jnp.stack([jnp.sum(u, axis=0), jnp.sum(u * u, axis=0)])


def _passA1(x_pad, parts, W1, b1, W2, b2):
    grid = N // NB
    return pl.pallas_call(
        _passA1_body,
        grid=(grid,),
        in_specs=[
            pl.BlockSpec((NB, L), lambda i: (i, 0)),
            pl.BlockSpec((2, NB, L), lambda i: (0, i, 0)),
            pl.BlockSpec((L, H), lambda i: (0, 0)),
            pl.BlockSpec((1, H), lambda i: (0, 0)),
            pl.BlockSpec((H, H), lambda i: (0, 0)),
            pl.BlockSpec((1, H), lambda i: (0, 0)),
        ],
        out_specs=[
            pl.BlockSpec((NB, H), lambda i: (i, 0)),
            pl.BlockSpec((2, H), lambda i: (0, 0)),
        ],
        out_shape=[
            jax.ShapeDtypeStruct((N, H), jnp.float32),
            jax.ShapeDtypeStruct((2, H), jnp.float32),
        ],
    )(x_pad, parts, W1, b1, W2, b2)


def _passA_body(h0_ref, h1_ref, h2_ref, h3_ref, ag_ref,
                W1_ref, b1_ref, W2_ref, b2_ref, u_ref, st_ref):
    i = pl.program_id(0)
    a = jnp.concatenate(
        [h0_ref[...] + ag_ref[0], h1_ref[...] + ag_ref[1],
         h2_ref[...] + ag_ref[2], h3_ref[...] + ag_ref[3]], axis=1)
    t = jnp.maximum(jnp.dot(a, W1_ref[...],
                            preferred_element_type=jnp.float32)
                    + b1_ref[...], 0.0)
    u = jnp.dot(t, W2_ref[...], preferred_element_type=jnp.float32) \
        + b2_ref[...]
    u_ref[...] = u

    @pl.when(i == 0)
    def _():
        st_ref[...] = jnp.zeros_like(st_ref)
    st_ref[...] += jnp.stack([jnp.sum(u, axis=0), jnp.sum(u * u, axis=0)])


def _passA(hs, aggr, W1, b1, W2, b2):
    grid = N // NB
    return pl.pallas_call(
        _passA_body,
        grid=(grid,),
        in_specs=[
            pl.BlockSpec((NB, L), lambda i: (i, 0)),
            pl.BlockSpec((NB, L), lambda i: (i, 0)),
            pl.BlockSpec((NB, L), lambda i: (i, 0)),
            pl.BlockSpec((NB, L), lambda i: (i, 0)),
            pl.BlockSpec((4, NB, L), lambda i: (0, i, 0)),
            pl.BlockSpec((H, H), lambda i: (0, 0)),
            pl.BlockSpec((1, H), lambda i: (0, 0)),
            pl.BlockSpec((H, H), lambda i: (0, 0)),
            pl.BlockSpec((1, H), lambda i: (0, 0)),
        ],
        out_specs=[
            pl.BlockSpec((NB, H), lambda i: (i, 0)),
            pl.BlockSpec((2, H), lambda i: (0, 0)),
        ],
        out_shape=[
            jax.ShapeDtypeStruct((N, H), jnp.float32),
            jax.ShapeDtypeStruct((2, H), jnp.float32),
        ],
    )(hs[0], hs[1], hs[2], hs[3], aggr, W1, b1, W2, b2)


def _passB_body(split, u_ref, st_ref, g_ref, b_ref, *out_refs):
    m = st_ref[0:1] * (1.0 / N)
    v = st_ref[1:2] * (1.0 / N) - m * m
    scale = g_ref[...] / jnp.sqrt(v + 1e-5)
    h = jnp.maximum((u_ref[...] - m) * scale + b_ref[...], 0.0)
    if split:
        for c in range(4):
            out_refs[c][...] = h[:, c * L:(c + 1) * L]
    else:
        out_refs[0][...] = h


def _passB(u, st, g, b, split):
    grid = N // NB
    if split:
        out_specs = [pl.BlockSpec((NB, L), lambda i: (i, 0))
                     for _ in range(4)]
        out_shape = [jax.ShapeDtypeStruct((N, L), jnp.float32)
                     for _ in range(4)]
    else:
        out_specs = [pl.BlockSpec((NB, H), lambda i: (i, 0))]
        out_shape = [jax.ShapeDtypeStruct((N, H), jnp.float32)]
    return pl.pallas_call(
        functools.partial(_passB_body, split),
        grid=(grid,),
        in_specs=[
            pl.BlockSpec((NB, H), lambda i: (i, 0)),
            pl.BlockSpec((2, H), lambda i: (0, 0)),
            pl.BlockSpec((1, H), lambda i: (0, 0)),
            pl.BlockSpec((1, H), lambda i: (0, 0)),
        ],
        out_specs=out_specs,
        out_shape=out_shape,
    )(u, st, g, b)


def _pool_body(h_ref, b_ref, sums_ref):
    i = pl.program_id(0)
    seg = b_ref[0]
    gid = lax.broadcasted_iota(jnp.int32, (NB, G), 1)
    onehot = jnp.where(seg == gid, 1.0, 0.0)
    h2 = jnp.concatenate(
        [h_ref[...], jnp.ones((NB, 1), jnp.float32)], axis=1)
    part = lax.dot_general(onehot, h2, (((0,), (0,)), ((), ())),
                           precision=lax.Precision.HIGHEST,
                           preferred_element_type=jnp.float32)

    @pl.when(i == 0)
    def _():
        sums_ref[...] = jnp.zeros_like(sums_ref)
    sums_ref[...] += part


def _pool(h, batch3):
    grid = N // NB
    return pl.pallas_call(
        _pool_body,
        grid=(grid,),
        in_specs=[
            pl.BlockSpec((NB, H), lambda i: (i, 0)),
            pl.BlockSpec((1, NB, 1), lambda i: (i, 0, 0)),
        ],
        out_specs=pl.BlockSpec((G, H + 1), lambda i: (0, 0)),
        out_shape=jax.ShapeDtypeStruct((G, H + 1), jnp.float32),
    )(h, batch3)


def _head_body(s_ref, W1_ref, b1_ref, W2_ref, b2_ref, o_ref):
    s = s_ref[...]
    pooled = s[:, :H] / jnp.maximum(s[:, H:H + 1], 1.0)
    z = jnp.maximum(jnp.dot(pooled, W1_ref[...],
                            preferred_element_type=jnp.float32)
                    + b1_ref[...], 0.0)
    o_ref[...] = jnp.dot(z, W2_ref[...],
                         preferred_element_type=jnp.float32) + b2_ref[...]


def _head(sums, W1, b1, W2, b2):
    return pl.pallas_call(
        _head_body,
        out_shape=jax.ShapeDtypeStruct((G, 1), jnp.float32),
    )(sums, W1, b1, W2, b2)


def kernel(x, edge_index, edge_attr, batch, params):
    src = edge_index[0].astype(jnp.int32)
    dst = edge_index[1].astype(jnp.int32)
    batch3 = batch.astype(jnp.int32).reshape(N // NB, NB, 1)

    x_pad = jnp.pad(x, ((0, 0), (0, L - 12)))
    eeW = jnp.pad(params['ee_W'], ((0, 0), (0, L - 12)))
    eeb = jnp.pad(params['ee_b'], (0, L - 12)).reshape(1, L)
    elW = jnp.pad(params['el_W'], ((0, L - 12), (0, 0)))
    elb = params['el_b'].reshape(1, H)
    m1W1 = jnp.pad(params['mlp1']['W1'], ((0, L - 12), (0, 0)))

    ea1, ead = _edge_prep(edge_attr, eeW, eeb, elW, elb)

    parts = _sc_gine_c1(x_pad, ea1.reshape(1, E, L), src, dst)
    u1, st1 = _passA1(x_pad, parts, m1W1,
                      params['mlp1']['b1'].reshape(1, H),
                      params['mlp1']['W2'],
                      params['mlp1']['b2'].reshape(1, H))
    h1s = _passB(u1, st1, params['bn1_g'].reshape(1, H),
                 params['bn1_b'].reshape(1, H), split=True)

    tab1 = jnp.concatenate(h1s, axis=0)
    aggr2 = _sc_gine_c4(tab1, ead, src, dst)
    u2, st2 = _passA(h1s, aggr2, params['mlp2']['W1'],
                     params['mlp2']['b1'].reshape(1, H),
                     params['mlp2']['W2'],
                     params['mlp2']['b2'].reshape(1, H))
    h2s = _passB(u2, st2, params['bn2_g'].reshape(1, H),
                 params['bn2_b'].reshape(1, H), split=True)

    tab2 = jnp.concatenate(h2s, axis=0)
    aggr3 = _sc_gine_c4(tab2, ead, src, dst)
    u3, st3 = _passA(h2s, aggr3, params['mlp3']['W1'],
                     params['mlp3']['b1'].reshape(1, H),
                     params['mlp3']['W2'],
                     params['mlp3']['b2'].reshape(1, H))
    (h3,) = _passB(u3, st3, params['bn3_g'].reshape(1, H),
                   params['bn3_b'].reshape(1, H), split=False)

    sums = _pool(h3, batch3)
    return _head(sums, params['h_W1'],
                 params['h_b1'].reshape(1, H // 2),
                 params['h_W2'], params['h_b2'].reshape(1, 1))


# speedup vs baseline: 2.3341x; 1.0391x over previous
import functools

import jax
import jax.numpy as jnp
from jax import lax
from jax.experimental import pallas as pl
from jax.experimental.pallas import tpu as pltpu
from jax.experimental.pallas import tpu_sc as plsc

N = 100000
E = 1600000
H = 64
G = 128
L = 16
EB = 8000
NB = 1000
SC_B = 400
NP = 102400
NPT = NP // 16


def _edge_prep_body(ea_ref, eeW_ref, eeb_ref, elW_ref, elb_ref,
                    ea1_ref, ead_ref):
    ea1 = jnp.dot(ea_ref[...], eeW_ref[...],
                  preferred_element_type=jnp.float32) + eeb_ref[...]
    ea1_ref[...] = ea1
    ead = jnp.dot(ea1, elW_ref[...],
                  preferred_element_type=jnp.float32) + elb_ref[...]
    ead_ref[...] = jnp.stack([ead[:, c * L:(c + 1) * L] for c in range(4)])


def _edge_prep(edge_attr, eeW, eeb, elW, elb):
    grid = E // EB
    return pl.pallas_call(
        _edge_prep_body,
        grid=(grid,),
        in_specs=[
            pl.BlockSpec((EB, 4), lambda i: (i, 0)),
            pl.BlockSpec((4, L), lambda i: (0, 0)),
            pl.BlockSpec((1, L), lambda i: (0, 0)),
            pl.BlockSpec((L, H), lambda i: (0, 0)),
            pl.BlockSpec((1, H), lambda i: (0, 0)),
        ],
        out_specs=[
            pl.BlockSpec((EB, L), lambda i: (i, 0)),
            pl.BlockSpec((4, EB, L), lambda i: (0, i, 0)),
        ],
        out_shape=[
            jax.ShapeDtypeStruct((E, L), jnp.float32),
            jax.ShapeDtypeStruct((4, E, L), jnp.float32),
        ],
    )(edge_attr, eeW, eeb, elW, elb)


def _make_sc_gine(C):
    mesh = plsc.VectorSubcoreMesh(core_axis_name="c", subcore_axis_name="s")
    n_out = 4 if C == 4 else 2
    Et = E // 16 if C == 4 else E // 32
    nblk = Et // SC_B
    npass = 2 if C == 4 else 1

    @functools.partial(
        pl.kernel,
        mesh=mesh,
        compiler_params=pltpu.CompilerParams(use_tc_tiling_on_sc=False),
        out_type=jax.ShapeDtypeStruct((n_out, NP, L), jnp.float32),
        scratch_types=[
            [pltpu.VMEM((SC_B,), jnp.int32) for _ in range(2)],
            [pltpu.VMEM((SC_B,), jnp.int32) for _ in range(2)],
            [pltpu.VMEM((SC_B, L), jnp.float32) for _ in range(2)],
            [pltpu.VMEM((SC_B, L), jnp.float32) for _ in range(2)],
            pltpu.VMEM_SHARED((NP + 8, L), jnp.float32),
            [pltpu.SemaphoreType.DMA for _ in range(4)],
        ],
    )
    def sc_gine(tab_hbm, ea_hbm, src_hbm, dst_hbm, out_hbm,
                src_v, dst_v, rows_v, ea_v, acc, sems):
        core = lax.axis_index("c")
        tile = lax.axis_index("s")
        row0 = tile * NPT

        def fetch(base, cidx, s):
            pltpu.sync_copy(src_hbm.at[pl.ds(base, SC_B)], src_v[s])
            pltpu.sync_copy(dst_hbm.at[pl.ds(base, SC_B)], dst_v[s])
            if C == 4:
                offv = jnp.full((L,), cidx * N, jnp.int32)

                def addoff(j, _):
                    sl = pl.ds(j * L, L)
                    src_v[s][sl] = src_v[s][sl] + offv
                    return 0
                lax.fori_loop(0, SC_B // L, addoff, 0)
            g = pltpu.async_copy(tab_hbm.at[src_v[s]], rows_v[s], sems[s])
            e = pltpu.async_copy(ea_hbm.at[cidx, pl.ds(base, SC_B)],
                                 ea_v[s], sems[2 + s])
            return g, e

        def drain(handles, s):
            g, e = handles
            g.wait()
            e.wait()

            def body(i, _):
                rows_v[s][i] = jnp.maximum(rows_v[s][i] + ea_v[s][i], 0.0)
                return 0
            lax.fori_loop(0, SC_B, body, 0, unroll=8)
            pltpu.sync_copy(rows_v[s], acc.at[dst_v[s]], add=True)

        for p in range(npass):
            if C == 4:
                cidx = core * 2 + p
                tile_base = tile * Et
            else:
                cidx = 0
                tile_base = core * (E // 2) + tile * Et

            def zfill(j, _):
                rows_v[0][j] = jnp.zeros((L,), jnp.float32)
                return 0
            lax.fori_loop(0, SC_B, zfill, 0)
            for k in range(NPT // SC_B):
                pltpu.sync_copy(rows_v[0], acc.at[pl.ds(row0 + k * SC_B,
                                                        SC_B)])
            plsc.subcore_barrier()

            def pair(g, _):
                b0 = tile_base + (2 * g) * SC_B
                b1 = b0 + SC_B
                h0 = fetch(b0, cidx, 0)
                h1 = fetch(b1, cidx, 1)
                drain(h0, 0)
                drain(h1, 1)
                return 0
            lax.fori_loop(0, nblk // 2, pair, 0)
            if nblk % 2:
                drain(fetch(tile_base + (nblk - 1) * SC_B, cidx, 0), 0)

            plsc.subcore_barrier()
            out_idx = cidx if C == 4 else core
            pltpu.sync_copy(acc.at[pl.ds(row0, NPT)],
                            out_hbm.at[out_idx, pl.ds(row0, NPT)])
            if p + 1 < npass:
                plsc.subcore_barrier()

    return sc_gine


_sc_gine_c1 = _make_sc_gine(1)
_sc_gine_c4 = _make_sc_gine(4)


def _passA1_body(x_ref, pa_ref, W1_ref, b1_ref, W2_ref, b2_ref,
                 u_ref, st_ref):
    i = pl.program_id(0)
    a = x_ref[...] + pa_ref[0] + pa_ref[1]
    t = jnp.maximum(jnp.dot(a, W1_ref[...],
                            preferred_element_type=jnp.float32)
                    + b1_ref[...], 0.0)
    u = jnp.dot(t, W2_ref[...], preferred_element_type=jnp.float32) \
        + b2_ref[...]
    u_ref[...] = u

    @pl.when(i == 0)
    def _():
        st_ref[...] = jnp.zeros_like(st_ref)
    st_ref[...] += jnp.stack([jnp.sum(u, axis=0), jnp.sum(u * u, axis=0)])


def _passA1(x_pad, parts, W1, b1, W2, b2):
    grid = N // NB
    return pl.pallas_call(
        _passA1_body,
        grid=(grid,),
        in_specs=[
            pl.BlockSpec((NB, L), lambda i: (i, 0)),
            pl.BlockSpec((2, NB, L), lambda i: (0, i, 0)),
            pl.BlockSpec((L, H), lambda i: (0, 0)),
            pl.BlockSpec((1, H), lambda i: (0, 0)),
            pl.BlockSpec((H, H), lambda i: (0, 0)),
            pl.BlockSpec((1, H), lambda i: (0, 0)),
        ],
        out_specs=[
            pl.BlockSpec((NB, H), lambda i: (i, 0)),
            pl.BlockSpec((2, H), lambda i: (0, 0)),
        ],
        out_shape=[
            jax.ShapeDtypeStruct((N, H), jnp.float32),
            jax.ShapeDtypeStruct((2, H), jnp.float32),
        ],
    )(x_pad, parts, W1, b1, W2, b2)


def _passA_body(h0_ref, h1_ref, h2_ref, h3_ref, ag_ref,
                W1_ref, b1_ref, W2_ref, b2_ref, u_ref, st_ref):
    i = pl.program_id(0)
    a = jnp.concatenate(
        [h0_ref[...] + ag_ref[0], h1_ref[...] + ag_ref[1],
         h2_ref[...] + ag_ref[2], h3_ref[...] + ag_ref[3]], axis=1)
    t = jnp.maximum(jnp.dot(a, W1_ref[...],
                            preferred_element_type=jnp.float32)
                    + b1_ref[...], 0.0)
    u = jnp.dot(t, W2_ref[...], preferred_element_type=jnp.float32) \
        + b2_ref[...]
    u_ref[...] = u

    @pl.when(i == 0)
    def _():
        st_ref[...] = jnp.zeros_like(st_ref)
    st_ref[...] += jnp.stack([jnp.sum(u, axis=0), jnp.sum(u * u, axis=0)])


def _passA(hs, aggr, W1, b1, W2, b2):
    grid = N // NB
    return pl.pallas_call(
        _passA_body,
        grid=(grid,),
        in_specs=[
            pl.BlockSpec((NB, L), lambda i: (i, 0)),
            pl.BlockSpec((NB, L), lambda i: (i, 0)),
            pl.BlockSpec((NB, L), lambda i: (i, 0)),
            pl.BlockSpec((NB, L), lambda i: (i, 0)),
            pl.BlockSpec((4, NB, L), lambda i: (0, i, 0)),
            pl.BlockSpec((H, H), lambda i: (0, 0)),
            pl.BlockSpec((1, H), lambda i: (0, 0)),
            pl.BlockSpec((H, H), lambda i: (0, 0)),
            pl.BlockSpec((1, H), lambda i: (0, 0)),
        ],
        out_specs=[
            pl.BlockSpec((NB, H), lambda i: (i, 0)),
            pl.BlockSpec((2, H), lambda i: (0, 0)),
        ],
        out_shape=[
            jax.ShapeDtypeStruct((N, H), jnp.float32),
            jax.ShapeDtypeStruct((2, H), jnp.float32),
        ],
    )(hs[0], hs[1], hs[2], hs[3], aggr, W1, b1, W2, b2)


def _passB_body(split, u_ref, st_ref, g_ref, b_ref, *out_refs):
    m = st_ref[0:1] * (1.0 / N)
    v = st_ref[1:2] * (1.0 / N) - m * m
    scale = g_ref[...] / jnp.sqrt(v + 1e-5)
    h = jnp.maximum((u_ref[...] - m) * scale + b_ref[...], 0.0)
    if split:
        for c in range(4):
            out_refs[c][...] = h[:, c * L:(c + 1) * L]
    else:
        out_refs[0][...] = h


def _passB(u, st, g, b, split):
    grid = N // NB
    if split:
        out_specs = [pl.BlockSpec((NB, L), lambda i: (i, 0))
                     for _ in range(4)]
        out_shape = [jax.ShapeDtypeStruct((N, L), jnp.float32)
                     for _ in range(4)]
    else:
        out_specs = [pl.BlockSpec((NB, H), lambda i: (i, 0))]
        out_shape = [jax.ShapeDtypeStruct((N, H), jnp.float32)]
    return pl.pallas_call(
        functools.partial(_passB_body, split),
        grid=(grid,),
        in_specs=[
            pl.BlockSpec((NB, H), lambda i: (i, 0)),
            pl.BlockSpec((2, H), lambda i: (0, 0)),
            pl.BlockSpec((1, H), lambda i: (0, 0)),
            pl.BlockSpec((1, H), lambda i: (0, 0)),
        ],
        out_specs=out_specs,
        out_shape=out_shape,
    )(u, st, g, b)


def _pool_body(h_ref, b_ref, sums_ref):
    i = pl.program_id(0)
    seg = b_ref[0]
    gid = lax.broadcasted_iota(jnp.int32, (NB, G), 1)
    onehot = jnp.where(seg == gid, 1.0, 0.0)
    h2 = jnp.concatenate(
        [h_ref[...], jnp.ones((NB, 1), jnp.float32)], axis=1)
    part = lax.dot_general(onehot, h2, (((0,), (0,)), ((), ())),
                           precision=lax.Precision.HIGHEST,
                           preferred_element_type=jnp.float32)

    @pl.when(i == 0)
    def _():
        sums_ref[...] = jnp.zeros_like(sums_ref)
    sums_ref[...] += part


def _pool(h, batch3):
    grid = N // NB
    return pl.pallas_call(
        _pool_body,
        grid=(grid,),
        in_specs=[
            pl.BlockSpec((NB, H), lambda i: (i, 0)),
            pl.BlockSpec((1, NB, 1), lambda i: (i, 0, 0)),
        ],
        out_specs=pl.BlockSpec((G, H + 1), lambda i: (0, 0)),
        out_shape=jax.ShapeDtypeStruct((G, H + 1), jnp.float32),
    )(h, batch3)


def _head_body(s_ref, W1_ref, b1_ref, W2_ref, b2_ref, o_ref):
    s = s_ref[...]
    pooled = s[:, :H] / jnp.maximum(s[:, H:H + 1], 1.0)
    z = jnp.maximum(jnp.dot(pooled, W1_ref[...],
                            preferred_element_type=jnp.float32)
                    + b1_ref[...], 0.0)
    o_ref[...] = jnp.dot(z, W2_ref[...],
                         preferred_element_type=jnp.float32) + b2_ref[...]


def _head(sums, W1, b1, W2, b2):
    return pl.pallas_call(
        _head_body,
        out_shape=jax.ShapeDtypeStruct((G, 1), jnp.float32),
    )(sums, W1, b1, W2, b2)


def kernel(x, edge_index, edge_attr, batch, params):
    src = edge_index[0].astype(jnp.int32)
    dst = edge_index[1].astype(jnp.int32)
    batch3 = batch.astype(jnp.int32).reshape(N // NB, NB, 1)

    x_pad = jnp.pad(x, ((0, 0), (0, L - 12)))
    eeW = jnp.pad(params['ee_W'], ((0, 0), (0, L - 12)))
    eeb = jnp.pad(params['ee_b'], (0, L - 12)).reshape(1, L)
    elW = jnp.pad(params['el_W'], ((0, L - 12), (0, 0)))
    elb = params['el_b'].reshape(1, H)
    m1W1 = jnp.pad(params['mlp1']['W1'], ((0, L - 12), (0, 0)))

    ea1, ead = _edge_prep(edge_attr, eeW, eeb, elW, elb)

    parts = _sc_gine_c1(x_pad, ea1.reshape(1, E, L), src, dst)
    u1, st1 = _passA1(x_pad, parts, m1W1,
                      params['mlp1']['b1'].reshape(1, H),
                      params['mlp1']['W2'],
                      params['mlp1']['b2'].reshape(1, H))
    h1s = _passB(u1, st1, params['bn1_g'].reshape(1, H),
                 params['bn1_b'].reshape(1, H), split=True)

    tab1 = jnp.concatenate(h1s, axis=0)
    aggr2 = _sc_gine_c4(tab1, ead, src, dst)
    u2, st2 = _passA(h1s, aggr2, params['mlp2']['W1'],
                     params['mlp2']['b1'].reshape(1, H),
                     params['mlp2']['W2'],
                     params['mlp2']['b2'].reshape(1, H))
    h2s = _passB(u2, st2, params['bn2_g'].reshape(1, H),
                 params['bn2_b'].reshape(1, H), split=True)

    tab2 = jnp.concatenate(h2s, axis=0)
    aggr3 = _sc_gine_c4(tab2, ead, src, dst)
    u3, st3 = _passA(h2s, aggr3, params['mlp3']['W1'],
                     params['mlp3']['b1'].reshape(1, H),
                     params['mlp3']['W2'],
                     params['mlp3']['b2'].reshape(1, H))
    (h3,) = _passB(u3, st3, params['bn3_g'].reshape(1, H),
                   params['bn3_b'].reshape(1, H), split=False)

    sums = _pool(h3, batch3)
    return _head(sums, params['h_W1'],
                 params['h_b1'].reshape(1, H // 2),
                 params['h_W2'], params['h_b2'].reshape(1, 1))


# speedup vs baseline: 2.7504x; 1.1784x over previous
import functools

import jax
import jax.numpy as jnp
from jax import lax
from jax.experimental import pallas as pl
from jax.experimental.pallas import tpu as pltpu
from jax.experimental.pallas import tpu_sc as plsc

N = 100000
E = 1600000
H = 64
G = 128
L = 16
EB = 8000
NB = 1000
SC_B = 400
NP = 102400
NPT = NP // 16


ER = 1000


def _edge_prep_body(ea_ref, W1_ref, b1_ref, W2_ref, b2_ref,
                    ea1_ref, ead_ref):
    ea1 = jnp.dot(ea_ref[...], W1_ref[...],
                  preferred_element_type=jnp.float32) + b1_ref[...]
    ea1_ref[...] = ea1
    ead = jnp.dot(ea1, W2_ref[...],
                  preferred_element_type=jnp.float32) + b2_ref[...]
    ead_ref[...] = jnp.stack(
        [ead[:, c * 512:(c + 1) * 512] for c in range(4)])


def _edge_prep(ea_packed, BW1, bb1, BW2, bb2):
    grid = (E // 32) // ER
    return pl.pallas_call(
        _edge_prep_body,
        grid=(grid,),
        in_specs=[
            pl.BlockSpec((ER, 128), lambda i: (i, 0)),
            pl.BlockSpec((128, 512), lambda i: (0, 0)),
            pl.BlockSpec((1, 512), lambda i: (0, 0)),
            pl.BlockSpec((512, 2048), lambda i: (0, 0)),
            pl.BlockSpec((1, 2048), lambda i: (0, 0)),
        ],
        out_specs=[
            pl.BlockSpec((ER, 512), lambda i: (i, 0)),
            pl.BlockSpec((4, ER, 512), lambda i: (0, i, 0)),
        ],
        out_shape=[
            jax.ShapeDtypeStruct((E // 32, 512), jnp.float32),
            jax.ShapeDtypeStruct((4, E // 32, 512), jnp.float32),
        ],
    )(ea_packed, BW1, bb1, BW2, bb2)


def _make_sc_gine(C):
    mesh = plsc.VectorSubcoreMesh(core_axis_name="c", subcore_axis_name="s")
    n_out = 4 if C == 4 else 2
    Et = E // 16 if C == 4 else E // 32
    nblk = Et // SC_B
    npass = 2 if C == 4 else 1

    @functools.partial(
        pl.kernel,
        mesh=mesh,
        compiler_params=pltpu.CompilerParams(use_tc_tiling_on_sc=False),
        out_type=jax.ShapeDtypeStruct((n_out, NP, L), jnp.float32),
        scratch_types=[
            [pltpu.VMEM((SC_B,), jnp.int32) for _ in range(2)],
            [pltpu.VMEM((SC_B,), jnp.int32) for _ in range(2)],
            [pltpu.VMEM((SC_B, L), jnp.float32) for _ in range(2)],
            [pltpu.VMEM((SC_B * L,), jnp.float32) for _ in range(2)],
            pltpu.VMEM_SHARED((NP + 8, L), jnp.float32),
            [pltpu.SemaphoreType.DMA for _ in range(4)],
        ],
    )
    def sc_gine(tab_hbm, ea_hbm, src_hbm, dst_hbm, out_hbm,
                src_v, dst_v, rows_v, ea_v, acc, sems):
        core = lax.axis_index("c")
        tile = lax.axis_index("s")
        row0 = tile * NPT

        def fetch(base, cidx, s):
            pltpu.sync_copy(src_hbm.at[pl.ds(base, SC_B)], src_v[s])
            pltpu.sync_copy(dst_hbm.at[pl.ds(base, SC_B)], dst_v[s])
            if C == 4:
                offv = jnp.full((L,), cidx * N, jnp.int32)

                def addoff(j, _):
                    sl = pl.ds(j * L, L)
                    src_v[s][sl] = src_v[s][sl] + offv
                    return 0
                lax.fori_loop(0, SC_B // L, addoff, 0)
            g = pltpu.async_copy(tab_hbm.at[src_v[s]], rows_v[s], sems[s])
            e = pltpu.async_copy(
                ea_hbm.at[cidx, pl.ds(base * L, SC_B * L)],
                ea_v[s], sems[2 + s])
            return g, e

        def drain(handles, s):
            g, e = handles
            g.wait()
            e.wait()

            def body(i, _):
                rows_v[s][i] = jnp.maximum(
                    rows_v[s][i] + ea_v[s][pl.ds(i * L, L)], 0.0)
                return 0
            lax.fori_loop(0, SC_B, body, 0, unroll=8)
            pltpu.sync_copy(rows_v[s], acc.at[dst_v[s]], add=True)

        for p in range(npass):
            if C == 4:
                cidx = core * 2 + p
                tile_base = tile * Et
            else:
                cidx = 0
                tile_base = core * (E // 2) + tile * Et

            def zfill(j, _):
                rows_v[0][j] = jnp.zeros((L,), jnp.float32)
                return 0
            lax.fori_loop(0, SC_B, zfill, 0)
            for k in range(NPT // SC_B):
                pltpu.sync_copy(rows_v[0], acc.at[pl.ds(row0 + k * SC_B,
                                                        SC_B)])
            plsc.subcore_barrier()

            def pair(g, _):
                b0 = tile_base + (2 * g) * SC_B
                b1 = b0 + SC_B
                h0 = fetch(b0, cidx, 0)
                h1 = fetch(b1, cidx, 1)
                drain(h0, 0)
                drain(h1, 1)
                return 0
            lax.fori_loop(0, nblk // 2, pair, 0)
            if nblk % 2:
                drain(fetch(tile_base + (nblk - 1) * SC_B, cidx, 0), 0)

            plsc.subcore_barrier()
            out_idx = cidx if C == 4 else core
            pltpu.sync_copy(acc.at[pl.ds(row0, NPT)],
                            out_hbm.at[out_idx, pl.ds(row0, NPT)])
            if p + 1 < npass:
                plsc.subcore_barrier()

    return sc_gine


_sc_gine_c1 = _make_sc_gine(1)
_sc_gine_c4 = _make_sc_gine(4)


def _passA1_body(x_ref, pa_ref, W1_ref, b1_ref, W2_ref, b2_ref,
                 u_ref, st_ref):
    i = pl.program_id(0)
    a = x_ref[...] + pa_ref[0] + pa_ref[1]
    t = jnp.maximum(jnp.dot(a, W1_ref[...],
                            preferred_element_type=jnp.float32)
                    + b1_ref[...], 0.0)
    u = jnp.dot(t, W2_ref[...], preferred_element_type=jnp.float32) \
        + b2_ref[...]
    u_ref[...] = u

    @pl.when(i == 0)
    def _():
        st_ref[...] = jnp.zeros_like(st_ref)
    st_ref[...] += jnp.stack([jnp.sum(u, axis=0), jnp.sum(u * u, axis=0)])


def _passA1(x_pad, parts, W1, b1, W2, b2):
    grid = N // NB
    return pl.pallas_call(
        _passA1_body,
        grid=(grid,),
        in_specs=[
            pl.BlockSpec((NB, L), lambda i: (i, 0)),
            pl.BlockSpec((2, NB, L), lambda i: (0, i, 0)),
            pl.BlockSpec((L, H), lambda i: (0, 0)),
            pl.BlockSpec((1, H), lambda i: (0, 0)),
            pl.BlockSpec((H, H), lambda i: (0, 0)),
            pl.BlockSpec((1, H), lambda i: (0, 0)),
        ],
        out_specs=[
            pl.BlockSpec((NB, H), lambda i: (i, 0)),
            pl.BlockSpec((2, H), lambda i: (0, 0)),
        ],
        out_shape=[
            jax.ShapeDtypeStruct((N, H), jnp.float32),
            jax.ShapeDtypeStruct((2, H), jnp.float32),
        ],
    )(x_pad, parts, W1, b1, W2, b2)


def _passA_body(h0_ref, h1_ref, h2_ref, h3_ref, ag_ref,
                W1_ref, b1_ref, W2_ref, b2_ref, u_ref, st_ref):
    i = pl.program_id(0)
    a = jnp.concatenate(
        [h0_ref[...] + ag_ref[0], h1_ref[...] + ag_ref[1],
         h2_ref[...] + ag_ref[2], h3_ref[...] + ag_ref[3]], axis=1)
    t = jnp.maximum(jnp.dot(a, W1_ref[...],
                            preferred_element_type=jnp.float32)
                    + b1_ref[...], 0.0)
    u = jnp.dot(t, W2_ref[...], preferred_element_type=jnp.float32) \
        + b2_ref[...]
    u_ref[...] = u

    @pl.when(i == 0)
    def _():
        st_ref[...] = jnp.zeros_like(st_ref)
    st_ref[...] += jnp.stack([jnp.sum(u, axis=0), jnp.sum(u * u, axis=0)])


def _passA(hs, aggr, W1, b1, W2, b2):
    grid = N // NB
    return pl.pallas_call(
        _passA_body,
        grid=(grid,),
        in_specs=[
            pl.BlockSpec((NB, L), lambda i: (i, 0)),
            pl.BlockSpec((NB, L), lambda i: (i, 0)),
            pl.BlockSpec((NB, L), lambda i: (i, 0)),
            pl.BlockSpec((NB, L), lambda i: (i, 0)),
            pl.BlockSpec((4, NB, L), lambda i: (0, i, 0)),
            pl.BlockSpec((H, H), lambda i: (0, 0)),
            pl.BlockSpec((1, H), lambda i: (0, 0)),
            pl.BlockSpec((H, H), lambda i: (0, 0)),
            pl.BlockSpec((1, H), lambda i: (0, 0)),
        ],
        out_specs=[
            pl.BlockSpec((NB, H), lambda i: (i, 0)),
            pl.BlockSpec((2, H), lambda i: (0, 0)),
        ],
        out_shape=[
            jax.ShapeDtypeStruct((N, H), jnp.float32),
            jax.ShapeDtypeStruct((2, H), jnp.float32),
        ],
    )(hs[0], hs[1], hs[2], hs[3], aggr, W1, b1, W2, b2)


def _passB_body(split, u_ref, st_ref, g_ref, b_ref, *out_refs):
    m = st_ref[0:1] * (1.0 / N)
    v = st_ref[1:2] * (1.0 / N) - m * m
    scale = g_ref[...] / jnp.sqrt(v + 1e-5)
    h = jnp.maximum((u_ref[...] - m) * scale + b_ref[...], 0.0)
    if split:
        for c in range(4):
            out_refs[c][...] = h[:, c * L:(c + 1) * L]
    else:
        out_refs[0][...] = h


def _passB(u, st, g, b, split):
    grid = N // NB
    if split:
        out_specs = [pl.BlockSpec((NB, L), lambda i: (i, 0))
                     for _ in range(4)]
        out_shape = [jax.ShapeDtypeStruct((N, L), jnp.float32)
                     for _ in range(4)]
    else:
        out_specs = [pl.BlockSpec((NB, H), lambda i: (i, 0))]
        out_shape = [jax.ShapeDtypeStruct((N, H), jnp.float32)]
    return pl.pallas_call(
        functools.partial(_passB_body, split),
        grid=(grid,),
        in_specs=[
            pl.BlockSpec((NB, H), lambda i: (i, 0)),
            pl.BlockSpec((2, H), lambda i: (0, 0)),
            pl.BlockSpec((1, H), lambda i: (0, 0)),
            pl.BlockSpec((1, H), lambda i: (0, 0)),
        ],
        out_specs=out_specs,
        out_shape=out_shape,
    )(u, st, g, b)


def _pool_body(h_ref, b_ref, sums_ref):
    i = pl.program_id(0)
    seg = b_ref[0]
    gid = lax.broadcasted_iota(jnp.int32, (NB, G), 1)
    onehot = jnp.where(seg == gid, 1.0, 0.0)
    h2 = jnp.concatenate(
        [h_ref[...], jnp.ones((NB, 1), jnp.float32)], axis=1)
    part = lax.dot_general(onehot, h2, (((0,), (0,)), ((), ())),
                           precision=lax.Precision.HIGHEST,
                           preferred_element_type=jnp.float32)

    @pl.when(i == 0)
    def _():
        sums_ref[...] = jnp.zeros_like(sums_ref)
    sums_ref[...] += part


def _pool(h, batch3):
    grid = N // NB
    return pl.pallas_call(
        _pool_body,
        grid=(grid,),
        in_specs=[
            pl.BlockSpec((NB, H), lambda i: (i, 0)),
            pl.BlockSpec((1, NB, 1), lambda i: (i, 0, 0)),
        ],
        out_specs=pl.BlockSpec((G, H + 1), lambda i: (0, 0)),
        out_shape=jax.ShapeDtypeStruct((G, H + 1), jnp.float32),
    )(h, batch3)


def _head_body(s_ref, W1_ref, b1_ref, W2_ref, b2_ref, o_ref):
    s = s_ref[...]
    pooled = s[:, :H] / jnp.maximum(s[:, H:H + 1], 1.0)
    z = jnp.maximum(jnp.dot(pooled, W1_ref[...],
                            preferred_element_type=jnp.float32)
                    + b1_ref[...], 0.0)
    o_ref[...] = jnp.dot(z, W2_ref[...],
                         preferred_element_type=jnp.float32) + b2_ref[...]


def _head(sums, W1, b1, W2, b2):
    return pl.pallas_call(
        _head_body,
        out_shape=jax.ShapeDtypeStruct((G, 1), jnp.float32),
    )(sums, W1, b1, W2, b2)


def kernel(x, edge_index, edge_attr, batch, params):
    src = edge_index[0].astype(jnp.int32)
    dst = edge_index[1].astype(jnp.int32)
    batch3 = batch.astype(jnp.int32).reshape(N // NB, NB, 1)

    x_pad = jnp.pad(x, ((0, 0), (0, L - 12)))
    eeW = jnp.pad(params['ee_W'], ((0, 0), (0, L - 12)))
    eeb = jnp.pad(params['ee_b'], (0, L - 12))
    elW = jnp.pad(params['el_W'], ((0, L - 12), (0, 0)))
    m1W1 = jnp.pad(params['mlp1']['W1'], ((0, L - 12), (0, 0)))

    eye32 = jnp.eye(32, dtype=jnp.float32)
    BW1 = jnp.kron(eye32, eeW)
    bb1 = jnp.tile(eeb, 32).reshape(1, 512)
    BW2 = jnp.concatenate(
        [jnp.kron(eye32, elW[:, c * L:(c + 1) * L]) for c in range(4)],
        axis=1)
    bb2 = jnp.concatenate(
        [jnp.tile(params['el_b'][c * L:(c + 1) * L], 32) for c in range(4)],
    ).reshape(1, 2048)

    ea1p, eadp = _edge_prep(edge_attr.reshape(E // 32, 128),
                            BW1, bb1, BW2, bb2)
    ea1_flat = ea1p.reshape(1, E * L)
    ead_flat = eadp.reshape(4, E * L)

    parts = _sc_gine_c1(x_pad, ea1_flat, src, dst)
    u1, st1 = _passA1(x_pad, parts, m1W1,
                      params['mlp1']['b1'].reshape(1, H),
                      params['mlp1']['W2'],
                      params['mlp1']['b2'].reshape(1, H))
    h1s = _passB(u1, st1, params['bn1_g'].reshape(1, H),
                 params['bn1_b'].reshape(1, H), split=True)

    tab1 = jnp.concatenate(h1s, axis=0)
    aggr2 = _sc_gine_c4(tab1, ead_flat, src, dst)
    u2, st2 = _passA(h1s, aggr2, params['mlp2']['W1'],
                     params['mlp2']['b1'].reshape(1, H),
                     params['mlp2']['W2'],
                     params['mlp2']['b2'].reshape(1, H))
    h2s = _passB(u2, st2, params['bn2_g'].reshape(1, H),
                 params['bn2_b'].reshape(1, H), split=True)

    tab2 = jnp.concatenate(h2s, axis=0)
    aggr3 = _sc_gine_c4(tab2, ead_flat, src, dst)
    u3, st3 = _passA(h2s, aggr3, params['mlp3']['W1'],
                     params['mlp3']['b1'].reshape(1, H),
                     params['mlp3']['W2'],
                     params['mlp3']['b2'].reshape(1, H))
    (h3,) = _passB(u3, st3, params['bn3_g'].reshape(1, H),
                   params['bn3_b'].reshape(1, H), split=False)

    sums = _pool(h3, batch3)
    return _head(sums, params['h_W1'],
                 params['h_b1'].reshape(1, H // 2),
                 params['h_W2'], params['h_b2'].reshape(1, 1))
